# baseline (speedup 1.0000x reference)
from functools import partial

import jax
import jax.numpy as jnp
from jax.experimental import pallas as pl
from jax.experimental.pallas import tpu as pltpu


def _rnn_body(x_ref, h0_ref, wih_ref, whh_ref, wout_ref, bias_ref, bout_ref,
              y_ref, h_s, xp_e, xp_o, hall_e, hall_o, *, TB, BB, NT2):
    k = pl.program_id(0)
    I = x_ref.shape[-1]
    H = whh_ref.shape[0]
    O = wout_ref.shape[-1]
    CH = 16
    TC = TB // CH
    SP = TB // CH
    a_pos = {c * SP: c for c in range(CH)}
    c_pos = {c * SP + SP // 2: c for c in range(CH)}
    whh = whh_ref[...]

    def a_chunk(c, x_off, xp_dst):
        xin = x_ref[x_off + c * TC:x_off + (c + 1) * TC].reshape(TC * BB, I)
        xp = jnp.dot(xin, wih_ref[...],
                     preferred_element_type=jnp.float32) + bias_ref[...]
        xp_dst[c * TC:(c + 1) * TC] = xp.reshape(TC, BB, H)

    def c_chunk(c, hall_src, y_off):
        hall = hall_src[c * TC:(c + 1) * TC].reshape(TC * BB, H)
        y = jnp.dot(hall, wout_ref[...],
                    preferred_element_type=jnp.float32) + bout_ref[...]
        y_ref[y_off + c * TC:y_off + (c + 1) * TC] = y.reshape(TC, BB, O)

    h = h_s[...]
    for i in range(TB):
        z = xp_o[i] + jnp.dot(h, whh, preferred_element_type=jnp.float32)
        if i in a_pos:
            a_chunk(a_pos[i], 0, xp_e)
        if i in c_pos:
            c_chunk(c_pos[i], hall_e, 0)
        h = jnp.tanh(z)
        hall_o[i] = h

    h = jnp.where(k == 0, h0_ref[...], h)

    for i in range(TB):
        z = xp_e[i] + jnp.dot(h, whh, preferred_element_type=jnp.float32)
        if i in a_pos:
            a_chunk(a_pos[i], TB, xp_o)
        if i in c_pos:
            c_chunk(c_pos[i], hall_o, TB)
        h = jnp.tanh(z)
        hall_e[i] = h
    h_s[...] = h


def kernel(x, h0, W_ih, b_ih, W_hh, b_hh, W_out, b_out):
    T, B, I = x.shape
    H = W_ih.shape[0]
    O = W_out.shape[0]
    TB = 32
    BB = B
    NT2 = T // (2 * TB)

    bias = (b_ih + b_hh).reshape(1, H)
    bout = b_out.reshape(1, O)

    return pl.pallas_call(
        partial(_rnn_body, TB=TB, BB=BB, NT2=NT2),
        grid=(NT2 + 1,),
        in_specs=[
            pl.BlockSpec((2 * TB, BB, I),
                         lambda k: (jnp.minimum(k, NT2 - 1), 0, 0)),
            pl.BlockSpec((BB, H), lambda k: (0, 0)),
            pl.BlockSpec((I, H), lambda k: (0, 0)),
            pl.BlockSpec((H, H), lambda k: (0, 0)),
            pl.BlockSpec((H, O), lambda k: (0, 0)),
            pl.BlockSpec((1, H), lambda k: (0, 0)),
            pl.BlockSpec((1, O), lambda k: (0, 0)),
        ],
        out_specs=pl.BlockSpec(
            (2 * TB, BB, O), lambda k: (jnp.maximum(k - 1, 0), 0, 0)),
        out_shape=jax.ShapeDtypeStruct((T, B, O), jnp.float32),
        scratch_shapes=[
            pltpu.VMEM((BB, H), jnp.float32),
            pltpu.VMEM((TB, BB, H), jnp.float32),
            pltpu.VMEM((TB, BB, H), jnp.float32),
            pltpu.VMEM((TB, BB, H), jnp.float32),
            pltpu.VMEM((TB, BB, H), jnp.float32),
        ],
        compiler_params=pltpu.CompilerParams(
            dimension_semantics=("arbitrary",),
            vmem_limit_bytes=56 * 1024 * 1024,
        ),
        name="elman_rnn_paired",
    )(x, h0, W_ih.T, W_hh.T, W_out.T, bias, bout)


# speedup vs baseline: 1.0068x; 1.0068x over previous
from functools import partial

import jax
import jax.numpy as jnp
from jax.experimental import pallas as pl
from jax.experimental.pallas import tpu as pltpu


def _rnn_body(x_ref, h0_ref, wih_ref, whh_ref, wout_ref, bias_ref, bout_ref,
              y_ref, h_s, xp_e, xp_o, hall_e, hall_o, *, TB, BB, NT2):
    k = pl.program_id(0)
    I = x_ref.shape[-1]
    H = whh_ref.shape[0]
    O = wout_ref.shape[-1]
    CH = 8
    TC = TB // CH
    SP = TB // CH
    a_pos = {c * SP: c for c in range(CH)}
    c_pos = {c * SP + SP // 2: c for c in range(CH)}
    whh = whh_ref[...]

    def a_chunk(c, x_off, xp_dst):
        xin = x_ref[x_off + c * TC:x_off + (c + 1) * TC].reshape(TC * BB, I)
        xp = jnp.dot(xin, wih_ref[...],
                     preferred_element_type=jnp.float32) + bias_ref[...]
        xp_dst[c * TC:(c + 1) * TC] = xp.reshape(TC, BB, H)

    def c_chunk(c, hall_src, y_off):
        hall = hall_src[c * TC:(c + 1) * TC].reshape(TC * BB, H)
        y = jnp.dot(hall, wout_ref[...],
                    preferred_element_type=jnp.float32) + bout_ref[...]
        y_ref[y_off + c * TC:y_off + (c + 1) * TC] = y.reshape(TC, BB, O)

    h = h_s[...]
    for i in range(TB):
        z = xp_o[i] + jnp.dot(h, whh, preferred_element_type=jnp.float32)
        if i in a_pos:
            a_chunk(a_pos[i], 0, xp_e)
        if i in c_pos:
            c_chunk(c_pos[i], hall_e, 0)
        h = jnp.tanh(z)
        hall_o[i] = h

    h = jnp.where(k == 0, h0_ref[...], h)

    for i in range(TB):
        z = xp_e[i] + jnp.dot(h, whh, preferred_element_type=jnp.float32)
        if i in a_pos:
            a_chunk(a_pos[i], TB, xp_o)
        if i in c_pos:
            c_chunk(c_pos[i], hall_o, TB)
        h = jnp.tanh(z)
        hall_e[i] = h
    h_s[...] = h


def kernel(x, h0, W_ih, b_ih, W_hh, b_hh, W_out, b_out):
    T, B, I = x.shape
    H = W_ih.shape[0]
    O = W_out.shape[0]
    TB = 16
    BB = B
    NT2 = T // (2 * TB)

    bias = (b_ih + b_hh).reshape(1, H)
    bout = b_out.reshape(1, O)

    return pl.pallas_call(
        partial(_rnn_body, TB=TB, BB=BB, NT2=NT2),
        grid=(NT2 + 1,),
        in_specs=[
            pl.BlockSpec((2 * TB, BB, I),
                         lambda k: (jnp.minimum(k, NT2 - 1), 0, 0)),
            pl.BlockSpec((BB, H), lambda k: (0, 0)),
            pl.BlockSpec((I, H), lambda k: (0, 0)),
            pl.BlockSpec((H, H), lambda k: (0, 0)),
            pl.BlockSpec((H, O), lambda k: (0, 0)),
            pl.BlockSpec((1, H), lambda k: (0, 0)),
            pl.BlockSpec((1, O), lambda k: (0, 0)),
        ],
        out_specs=pl.BlockSpec(
            (2 * TB, BB, O), lambda k: (jnp.maximum(k - 1, 0), 0, 0)),
        out_shape=jax.ShapeDtypeStruct((T, B, O), jnp.float32),
        scratch_shapes=[
            pltpu.VMEM((BB, H), jnp.float32),
            pltpu.VMEM((TB, BB, H), jnp.float32),
            pltpu.VMEM((TB, BB, H), jnp.float32),
            pltpu.VMEM((TB, BB, H), jnp.float32),
            pltpu.VMEM((TB, BB, H), jnp.float32),
        ],
        compiler_params=pltpu.CompilerParams(
            dimension_semantics=("arbitrary",),
            vmem_limit_bytes=50 * 1024 * 1024,
        ),
        name="elman_rnn_paired",
    )(x, h0, W_ih.T, W_hh.T, W_out.T, bias, bout)
